# Initial kernel scaffold; baseline (speedup 1.0000x reference)
import jax
import jax.numpy as jnp
from jax.experimental import pallas as pl


def kernel(x, W, b, space_table, nan_table, pe):
    raise NotImplementedError("write your pallas kernel here")


# speedup vs baseline: 2.6176x; 2.6176x over previous
import functools

import jax
import jax.numpy as jnp
from jax import lax
from jax.experimental import pallas as pl
from jax.experimental.pallas import tpu as pltpu
from jax.experimental.pallas import tpu_sc as plsc

NC, NS = 2, 16
NW = NC * NS
T = 50
NTOK = 25
DM = 128
NTOKENS = T * NTOK
CH_T = 5
CHUNK = CH_T * NTOK
NCHUNK = NTOKENS // CHUNK
KV = DM // 16


def kernel(x, W, b, space_table, nan_table, pe):
    B = x.shape[0]
    assert B % NW == 0
    bpw = B // NW
    xf = x.reshape(B, NTOKENS * 3)
    Wt = W.T

    mesh = plsc.VectorSubcoreMesh(core_axis_name="c", subcore_axis_name="s")

    @functools.partial(
        pl.kernel,
        out_type=jax.ShapeDtypeStruct((B, NTOKENS, DM), jnp.float32),
        mesh=mesh,
        scratch_types=[
            pltpu.VMEM((NTOKENS * 3,), jnp.float32),
            pltpu.VMEM((3, DM), jnp.float32),
            pltpu.VMEM((T, DM), jnp.float32),
            pltpu.VMEM((NTOK, DM), jnp.float32),
            pltpu.VMEM((NTOK, DM), jnp.float32),
            pltpu.VMEM((2, DM), jnp.float32),
            pltpu.VMEM((DM,), jnp.float32),
            pltpu.VMEM((DM,), jnp.float32),
            pltpu.VMEM((2, CHUNK, DM), jnp.float32),
            pltpu.SemaphoreType.DMA((2,)),
        ],
        compiler_params=pltpu.CompilerParams(use_tc_tiling_on_sc=False,
                                             needs_layout_passes=False),
    )
    def emb_kernel(xf_hbm, wt_hbm, b_hbm, sp_hbm, nan_hbm, pe_hbm, out_hbm,
                   xv, wt_v, pe_v, sp_v, sp2_v, nan_v, b_v, nd_v, obuf, sems):
        wid = lax.axis_index("s") * NC + lax.axis_index("c")

        pltpu.sync_copy(wt_hbm, wt_v)
        pltpu.sync_copy(pe_hbm, pe_v)
        pltpu.sync_copy(sp_hbm, sp_v)
        pltpu.sync_copy(nan_hbm, nan_v)
        pltpu.sync_copy(b_hbm, b_v)

        def prow(j, carry):
            for k in range(KV):
                s = pl.ds(k * 16, 16)
                sp2_v[j, s] = sp_v[j, s] + b_v[s] + nan_v[0, s]
            return carry
        lax.fori_loop(0, NTOK, prow, 0)
        for k in range(KV):
            s = pl.ds(k * 16, 16)
            nd_v[s] = nan_v[1, s] - nan_v[0, s]

        zero16 = jnp.zeros((16,), jnp.float32)
        one16 = jnp.ones((16,), jnp.float32)

        def chunk_iter(it, carry):
            bi = it // NCHUNK
            c = it % NCHUNK
            bidx = wid * bpw + bi
            p = it % 2

            @pl.when(c == 0)
            def _():
                pltpu.sync_copy(xf_hbm.at[bidx], xv)

            @pl.when(it >= 2)
            def _():
                pltpu.make_async_copy(
                    obuf.at[p], out_hbm.at[0, pl.ds(0, CHUNK)], sems.at[p]
                ).wait()

            wv = tuple(wt_v[d, pl.ds(k * 16, 16)]
                       for d in range(3) for k in range(KV))
            ndv = tuple(nd_v[pl.ds(k * 16, 16)] for k in range(KV))

            def tok_body(tj, tcarry):
                wv, ndv = tcarry
                t = c * CH_T + tj // NTOK
                j = tj % NTOK
                g = c * CHUNK + tj
                i0 = jnp.full((16,), g * 3, jnp.int32)
                x0 = plsc.load_gather(xv, [i0])
                x1 = plsc.load_gather(xv, [i0 + 1])
                x2 = plsc.load_gather(xv, [i0 + 2])
                m0 = x0 != x0
                m1 = x1 != x1
                m2 = x2 != x2
                x0c = jnp.where(m0, zero16, x0)
                x1c = jnp.where(m1, zero16, x1)
                x2c = jnp.where(m2, zero16, x2)
                flag = jnp.where(m0 | m1 | m2, one16, zero16)
                for k in range(KV):
                    s = pl.ds(k * 16, 16)
                    o = (pe_v[t, s] + sp2_v[j, s]
                         + x0c * wv[k] + x1c * wv[KV + k] + x2c * wv[2 * KV + k]
                         + flag * ndv[k])
                    obuf[p, tj, s] = o
                return tcarry

            lax.fori_loop(0, CHUNK, tok_body, (wv, ndv))

            pltpu.async_copy(
                obuf.at[p], out_hbm.at[bidx, pl.ds(c * CHUNK, CHUNK)], sems.at[p]
            )
            return carry

        lax.fori_loop(0, bpw * NCHUNK, chunk_iter, 0)

        for p in range(2):
            pltpu.make_async_copy(
                obuf.at[p], out_hbm.at[0, pl.ds(0, CHUNK)], sems.at[p]
            ).wait()

    return emb_kernel(xf, Wt, b, space_table, nan_table, pe)
